# baseline (speedup 1.0000x reference)
import functools

import jax
import jax.numpy as jnp
from jax import lax
from jax.experimental import pallas as pl
from jax.experimental.pallas import tpu as pltpu
from jax.experimental.pallas import tpu_sc as plsc

_N_E = 8192
_DIM = 32
_BETA = 0.25
_PIX = 1024
_CHUNK = 2048
_NCHUNK = _N_E // _CHUNK


def _argmin_body(z_ref, cb_ref, idx_ref, dmin_ref):
    z2 = z_ref[0]
    znorm = jnp.sum(z2 * z2, axis=0, keepdims=True)

    best_v = None
    best_i = None
    for k in range(_NCHUNK):
        cbk = cb_ref[pl.ds(k * _CHUNK, _CHUNK), :]
        cnorm = jnp.sum(cbk * cbk, axis=1, keepdims=True)
        s = lax.dot_general(
            cbk, z2, (((1,), (0,)), ((), ())),
            precision=lax.Precision.HIGHEST,
            preferred_element_type=jnp.float32)
        d = (znorm + cnorm) - 2.0 * s
        mv = jnp.min(d, axis=0, keepdims=True)
        rows = lax.broadcasted_iota(jnp.int32, (_CHUNK, _PIX), 0) + (k * _CHUNK)
        mi = jnp.min(jnp.where(d == mv, rows, jnp.int32(2 ** 30)),
                     axis=0, keepdims=True)
        if best_v is None:
            best_v, best_i = mv, mi
        else:
            upd = mv < best_v
            best_v = jnp.where(upd, mv, best_v)
            best_i = jnp.where(upd, mi, best_i)

    idx_ref[...] = best_i[None]
    dmin_ref[...] = jnp.broadcast_to(jnp.sum(best_v), (1, 1, _PIX))


def _tc_argmin(zr, codebook):
    return pl.pallas_call(
        _argmin_body,
        grid=(zr.shape[0],),
        in_specs=[
            pl.BlockSpec((1, _DIM, _PIX), lambda b: (b, 0, 0)),
            pl.BlockSpec((_N_E, _DIM), lambda b: (0, 0)),
        ],
        out_specs=[
            pl.BlockSpec((1, 1, _PIX), lambda b: (b, 0, 0)),
            pl.BlockSpec((1, 1, _PIX), lambda b: (b, 0, 0)),
        ],
        out_shape=[
            jax.ShapeDtypeStruct((zr.shape[0], 1, _PIX), jnp.int32),
            jax.ShapeDtypeStruct((zr.shape[0], 1, _PIX), jnp.float32),
        ],
    )(zr, codebook)


_SC_LANES = 128


def _sc_gather(table, idx2d, n_rows):
    row_w = table.shape[1]
    info = plsc.get_sparse_core_info()
    nc, ns = info.num_cores, info.num_subcores
    nw = nc * ns
    rows_per_w = n_rows // nw
    chunks_per_w = rows_per_w // _SC_LANES
    mesh = plsc.VectorSubcoreMesh(core_axis_name="c", subcore_axis_name="s")

    @functools.partial(
        pl.kernel,
        mesh=mesh,
        out_type=jax.ShapeDtypeStruct((n_rows, row_w), jnp.float32),
        scratch_types=[
            pltpu.VMEM((chunks_per_w, _SC_LANES), jnp.int32),
            pltpu.VMEM((rows_per_w, row_w), jnp.float32),
            pltpu.SemaphoreType.DMA,
        ],
    )
    def gather(idx_hbm, table_hbm, out_hbm, idx_v, rows_v, sem):
        wid = lax.axis_index("s") * nc + lax.axis_index("c")
        base_chunk = wid * chunks_per_w
        pltpu.sync_copy(idx_hbm.at[pl.ds(base_chunk, chunks_per_w)], idx_v)
        copies = []
        for j in range(chunks_per_w):
            copies.append(pltpu.async_copy(
                table_hbm.at[idx_v.at[j]],
                rows_v.at[pl.ds(j * _SC_LANES, _SC_LANES)],
                sem))
        for c in copies:
            c.wait()
        pltpu.sync_copy(rows_v, out_hbm.at[pl.ds(wid * rows_per_w, rows_per_w)])

    return gather(idx2d, table)


def _transpose_body(q_ref, out_ref):
    out_ref[0] = q_ref[0][:, :_DIM].T


def _tc_transpose(q3):
    return pl.pallas_call(
        _transpose_body,
        grid=(q3.shape[0],),
        in_specs=[pl.BlockSpec((1, _PIX, _SC_LANES), lambda b: (b, 0, 0))],
        out_specs=pl.BlockSpec((1, _DIM, _PIX), lambda b: (b, 0, 0)),
        out_shape=jax.ShapeDtypeStruct((q3.shape[0], _DIM, _PIX), jnp.float32),
    )(q3)


def kernel(z, codebook):
    b, c, h, w = z.shape
    n_rows = b * h * w
    zr = z.reshape(b, c, h * w)

    idx, dsum = _tc_argmin(zr, codebook)

    cbp = jnp.pad(codebook, ((0, 0), (0, _SC_LANES - _DIM)))
    q = _sc_gather(cbp, idx.reshape(n_rows // _SC_LANES, _SC_LANES), n_rows)

    loss = (1.0 + _BETA) * jnp.sum(dsum[:, 0, 0]) / (n_rows * _DIM)

    qt = _tc_transpose(q.reshape(b, h * w, _SC_LANES))
    return loss, qt.reshape(b, c, h, w)


# speedup vs baseline: 1.0445x; 1.0445x over previous
import functools

import jax
import jax.numpy as jnp
from jax import lax
from jax.experimental import pallas as pl
from jax.experimental.pallas import tpu as pltpu
from jax.experimental.pallas import tpu_sc as plsc

_N_E = 8192
_DIM = 32
_BETA = 0.25
_PIX = 1024
_CHUNK = 2048
_NCHUNK = _N_E // _CHUNK


def _argmin_body(z_ref, cb_ref, idx_ref, dmin_ref):
    z2 = z_ref[0]
    znorm = jnp.sum(z2 * z2, axis=0, keepdims=True)

    best_v = None
    best_i = None
    for k in range(_NCHUNK):
        cbk = cb_ref[pl.ds(k * _CHUNK, _CHUNK), :]
        cnorm = jnp.sum(cbk * cbk, axis=1, keepdims=True)
        s = lax.dot_general(
            cbk, z2, (((1,), (0,)), ((), ())),
            precision=lax.Precision.HIGHEST,
            preferred_element_type=jnp.float32)
        d = cnorm - 2.0 * s
        mv = jnp.min(d, axis=0, keepdims=True)
        rows = lax.broadcasted_iota(jnp.int32, (_CHUNK, _PIX), 0) + (k * _CHUNK)
        mi = jnp.min(jnp.where(d == mv, rows, jnp.int32(2 ** 30)),
                     axis=0, keepdims=True)
        if best_v is None:
            best_v, best_i = mv, mi
        else:
            upd = mv < best_v
            best_v = jnp.where(upd, mv, best_v)
            best_i = jnp.where(upd, mi, best_i)

    idx_ref[...] = best_i[None]
    dmin_ref[...] = jnp.broadcast_to(
        jnp.sum(best_v + znorm), (1, 1, _PIX))


def _tc_argmin(zr, codebook):
    return pl.pallas_call(
        _argmin_body,
        grid=(zr.shape[0],),
        in_specs=[
            pl.BlockSpec((1, _DIM, _PIX), lambda b: (b, 0, 0)),
            pl.BlockSpec((_N_E, _DIM), lambda b: (0, 0)),
        ],
        out_specs=[
            pl.BlockSpec((1, 1, _PIX), lambda b: (b, 0, 0)),
            pl.BlockSpec((1, 1, _PIX), lambda b: (b, 0, 0)),
        ],
        out_shape=[
            jax.ShapeDtypeStruct((zr.shape[0], 1, _PIX), jnp.int32),
            jax.ShapeDtypeStruct((zr.shape[0], 1, _PIX), jnp.float32),
        ],
    )(zr, codebook)


_SC_LANES = 128


def _sc_gather(table, idx2d, n_rows):
    row_w = table.shape[1]
    info = plsc.get_sparse_core_info()
    nc, ns = info.num_cores, info.num_subcores
    nw = nc * ns
    rows_per_w = n_rows // nw
    chunks_per_w = rows_per_w // _SC_LANES
    mesh = plsc.VectorSubcoreMesh(core_axis_name="c", subcore_axis_name="s")

    @functools.partial(
        pl.kernel,
        mesh=mesh,
        out_type=jax.ShapeDtypeStruct((n_rows, row_w), jnp.float32),
        scratch_types=[
            pltpu.VMEM((chunks_per_w, _SC_LANES), jnp.int32),
            pltpu.VMEM((rows_per_w, row_w), jnp.float32),
            pltpu.SemaphoreType.DMA,
        ],
    )
    def gather(idx_hbm, table_hbm, out_hbm, idx_v, rows_v, sem):
        wid = lax.axis_index("s") * nc + lax.axis_index("c")
        base_chunk = wid * chunks_per_w
        pltpu.sync_copy(idx_hbm.at[pl.ds(base_chunk, chunks_per_w)], idx_v)
        copies = []
        for j in range(chunks_per_w):
            copies.append(pltpu.async_copy(
                table_hbm.at[idx_v.at[j]],
                rows_v.at[pl.ds(j * _SC_LANES, _SC_LANES)],
                sem))
        for c in copies:
            c.wait()
        pltpu.sync_copy(rows_v, out_hbm.at[pl.ds(wid * rows_per_w, rows_per_w)])

    return gather(idx2d, table)


def _transpose_body(q_ref, out_ref):
    out_ref[0] = q_ref[0][:, :_DIM].T


def _tc_transpose(q3):
    return pl.pallas_call(
        _transpose_body,
        grid=(q3.shape[0],),
        in_specs=[pl.BlockSpec((1, _PIX, _SC_LANES), lambda b: (b, 0, 0))],
        out_specs=pl.BlockSpec((1, _DIM, _PIX), lambda b: (b, 0, 0)),
        out_shape=jax.ShapeDtypeStruct((q3.shape[0], _DIM, _PIX), jnp.float32),
    )(q3)


def kernel(z, codebook):
    b, c, h, w = z.shape
    n_rows = b * h * w
    zr = z.reshape(b, c, h * w)

    idx, dsum = _tc_argmin(zr, codebook)

    cbp = jnp.pad(codebook, ((0, 0), (0, _SC_LANES - _DIM)))
    q = _sc_gather(cbp, idx.reshape(n_rows // _SC_LANES, _SC_LANES), n_rows)

    loss = (1.0 + _BETA) * jnp.sum(dsum[:, 0, 0]) / (n_rows * _DIM)

    qt = _tc_transpose(q.reshape(b, h * w, _SC_LANES))
    return loss, qt.reshape(b, c, h, w)


# speedup vs baseline: 2.4861x; 2.3801x over previous
import functools

import jax
import jax.numpy as jnp
from jax import lax
from jax.experimental import pallas as pl
from jax.experimental.pallas import tpu as pltpu
from jax.experimental.pallas import tpu_sc as plsc

_N_E = 8192
_DIM = 32
_BETA = 0.25
_PIX = 1024
_CHUNK = 2048
_NCHUNK = _N_E // _CHUNK


def _argmin_body(z_ref, cb_ref, idx_ref, dmin_ref):
    z2 = z_ref[0]
    znorm = jnp.sum(z2 * z2, axis=0, keepdims=True)

    best_v = None
    best_i = None
    for k in range(_NCHUNK):
        cbk = cb_ref[pl.ds(k * _CHUNK, _CHUNK), :]
        cnorm = jnp.sum(cbk * cbk, axis=1, keepdims=True)
        s = lax.dot_general(
            cbk, z2, (((1,), (0,)), ((), ())),
            precision=lax.Precision.DEFAULT,
            preferred_element_type=jnp.float32)
        d = cnorm - 2.0 * s
        mv = jnp.min(d, axis=0, keepdims=True)
        rows = lax.broadcasted_iota(jnp.int32, (_CHUNK, _PIX), 0) + (k * _CHUNK)
        mi = jnp.min(jnp.where(d == mv, rows, jnp.int32(2 ** 30)),
                     axis=0, keepdims=True)
        if best_v is None:
            best_v, best_i = mv, mi
        else:
            upd = mv < best_v
            best_v = jnp.where(upd, mv, best_v)
            best_i = jnp.where(upd, mi, best_i)

    idx_ref[...] = best_i[None]
    dmin_ref[...] = jnp.broadcast_to(
        jnp.sum(best_v + znorm), (1, 1, _PIX))


def _tc_argmin(zr, codebook):
    return pl.pallas_call(
        _argmin_body,
        grid=(zr.shape[0],),
        in_specs=[
            pl.BlockSpec((1, _DIM, _PIX), lambda b: (b, 0, 0)),
            pl.BlockSpec((_N_E, _DIM), lambda b: (0, 0)),
        ],
        out_specs=[
            pl.BlockSpec((1, 1, _PIX), lambda b: (b, 0, 0)),
            pl.BlockSpec((1, 1, _PIX), lambda b: (b, 0, 0)),
        ],
        out_shape=[
            jax.ShapeDtypeStruct((zr.shape[0], 1, _PIX), jnp.int32),
            jax.ShapeDtypeStruct((zr.shape[0], 1, _PIX), jnp.float32),
        ],
    )(zr, codebook)


_SC_LANES = 128


def _sc_gather(table, idx2d, n_rows):
    row_w = table.shape[1]
    info = plsc.get_sparse_core_info()
    nc, ns = info.num_cores, info.num_subcores
    nw = nc * ns
    rows_per_w = n_rows // nw
    chunks_per_w = rows_per_w // _SC_LANES
    mesh = plsc.VectorSubcoreMesh(core_axis_name="c", subcore_axis_name="s")

    @functools.partial(
        pl.kernel,
        mesh=mesh,
        out_type=jax.ShapeDtypeStruct((n_rows, row_w), jnp.float32),
        scratch_types=[
            pltpu.VMEM((chunks_per_w, _SC_LANES), jnp.int32),
            pltpu.VMEM((rows_per_w, row_w), jnp.float32),
            pltpu.SemaphoreType.DMA,
        ],
    )
    def gather(idx_hbm, table_hbm, out_hbm, idx_v, rows_v, sem):
        wid = lax.axis_index("s") * nc + lax.axis_index("c")
        base_chunk = wid * chunks_per_w
        pltpu.sync_copy(idx_hbm.at[pl.ds(base_chunk, chunks_per_w)], idx_v)
        copies = []
        for j in range(chunks_per_w):
            copies.append(pltpu.async_copy(
                table_hbm.at[idx_v.at[j]],
                rows_v.at[pl.ds(j * _SC_LANES, _SC_LANES)],
                sem))
        for c in copies:
            c.wait()
        pltpu.sync_copy(rows_v, out_hbm.at[pl.ds(wid * rows_per_w, rows_per_w)])

    return gather(idx2d, table)


def _transpose_body(q_ref, out_ref):
    out_ref[0] = q_ref[0][:, :_DIM].T


def _tc_transpose(q3):
    return pl.pallas_call(
        _transpose_body,
        grid=(q3.shape[0],),
        in_specs=[pl.BlockSpec((1, _PIX, _SC_LANES), lambda b: (b, 0, 0))],
        out_specs=pl.BlockSpec((1, _DIM, _PIX), lambda b: (b, 0, 0)),
        out_shape=jax.ShapeDtypeStruct((q3.shape[0], _DIM, _PIX), jnp.float32),
    )(q3)


def kernel(z, codebook):
    b, c, h, w = z.shape
    n_rows = b * h * w
    zr = z.reshape(b, c, h * w)

    idx, dsum = _tc_argmin(zr, codebook)

    cbp = jnp.pad(codebook, ((0, 0), (0, _SC_LANES - _DIM)))
    q = _sc_gather(cbp, idx.reshape(n_rows // _SC_LANES, _SC_LANES), n_rows)

    loss = (1.0 + _BETA) * jnp.sum(dsum[:, 0, 0]) / (n_rows * _DIM)

    qt = _tc_transpose(q.reshape(b, h * w, _SC_LANES))
    return loss, qt.reshape(b, c, h, w)


# speedup vs baseline: 2.4908x; 1.0019x over previous
import functools

import jax
import jax.numpy as jnp
from jax import lax
from jax.experimental import pallas as pl
from jax.experimental.pallas import tpu as pltpu
from jax.experimental.pallas import tpu_sc as plsc

_N_E = 8192
_DIM = 32
_BETA = 0.25
_PIX = 1024
_CHUNK = 4096
_NCHUNK = _N_E // _CHUNK


def _argmin_body(z_ref, cb_ref, idx_ref, dmin_ref):
    z2 = z_ref[0]
    znorm = jnp.sum(z2 * z2, axis=0, keepdims=True)

    best_v = None
    best_i = None
    for k in range(_NCHUNK):
        cbk = cb_ref[pl.ds(k * _CHUNK, _CHUNK), :]
        cnorm = jnp.sum(cbk * cbk, axis=1, keepdims=True)
        s = lax.dot_general(
            cbk, z2, (((1,), (0,)), ((), ())),
            precision=lax.Precision.DEFAULT,
            preferred_element_type=jnp.float32)
        d = cnorm - 2.0 * s
        mv = jnp.min(d, axis=0, keepdims=True)
        rows = lax.broadcasted_iota(jnp.int32, (_CHUNK, _PIX), 0) + (k * _CHUNK)
        mi = jnp.min(jnp.where(d == mv, rows, jnp.int32(2 ** 30)),
                     axis=0, keepdims=True)
        if best_v is None:
            best_v, best_i = mv, mi
        else:
            upd = mv < best_v
            best_v = jnp.where(upd, mv, best_v)
            best_i = jnp.where(upd, mi, best_i)

    idx_ref[...] = best_i[None]
    dmin_ref[...] = jnp.broadcast_to(
        jnp.sum(best_v + znorm), (1, 1, _PIX))


def _tc_argmin(zr, codebook):
    return pl.pallas_call(
        _argmin_body,
        grid=(zr.shape[0],),
        in_specs=[
            pl.BlockSpec((1, _DIM, _PIX), lambda b: (b, 0, 0)),
            pl.BlockSpec((_N_E, _DIM), lambda b: (0, 0)),
        ],
        out_specs=[
            pl.BlockSpec((1, 1, _PIX), lambda b: (b, 0, 0)),
            pl.BlockSpec((1, 1, _PIX), lambda b: (b, 0, 0)),
        ],
        out_shape=[
            jax.ShapeDtypeStruct((zr.shape[0], 1, _PIX), jnp.int32),
            jax.ShapeDtypeStruct((zr.shape[0], 1, _PIX), jnp.float32),
        ],
    )(zr, codebook)


_SC_LANES = 128


def _sc_gather(table, idx2d, n_rows):
    row_w = table.shape[1]
    info = plsc.get_sparse_core_info()
    nc, ns = info.num_cores, info.num_subcores
    nw = nc * ns
    rows_per_w = n_rows // nw
    chunks_per_w = rows_per_w // _SC_LANES
    mesh = plsc.VectorSubcoreMesh(core_axis_name="c", subcore_axis_name="s")

    @functools.partial(
        pl.kernel,
        mesh=mesh,
        out_type=jax.ShapeDtypeStruct((n_rows, row_w), jnp.float32),
        scratch_types=[
            pltpu.VMEM((chunks_per_w, _SC_LANES), jnp.int32),
            pltpu.VMEM((rows_per_w, row_w), jnp.float32),
            pltpu.SemaphoreType.DMA,
        ],
    )
    def gather(idx_hbm, table_hbm, out_hbm, idx_v, rows_v, sem):
        wid = lax.axis_index("s") * nc + lax.axis_index("c")
        base_chunk = wid * chunks_per_w
        pltpu.sync_copy(idx_hbm.at[pl.ds(base_chunk, chunks_per_w)], idx_v)
        copies = []
        for j in range(chunks_per_w):
            copies.append(pltpu.async_copy(
                table_hbm.at[idx_v.at[j]],
                rows_v.at[pl.ds(j * _SC_LANES, _SC_LANES)],
                sem))
        for c in copies:
            c.wait()
        pltpu.sync_copy(rows_v, out_hbm.at[pl.ds(wid * rows_per_w, rows_per_w)])

    return gather(idx2d, table)


def _transpose_body(q_ref, out_ref):
    out_ref[0] = q_ref[0][:, :_DIM].T


def _tc_transpose(q3):
    return pl.pallas_call(
        _transpose_body,
        grid=(q3.shape[0],),
        in_specs=[pl.BlockSpec((1, _PIX, _SC_LANES), lambda b: (b, 0, 0))],
        out_specs=pl.BlockSpec((1, _DIM, _PIX), lambda b: (b, 0, 0)),
        out_shape=jax.ShapeDtypeStruct((q3.shape[0], _DIM, _PIX), jnp.float32),
    )(q3)


def kernel(z, codebook):
    b, c, h, w = z.shape
    n_rows = b * h * w
    zr = z.reshape(b, c, h * w)

    idx, dsum = _tc_argmin(zr, codebook)

    cbp = jnp.pad(codebook, ((0, 0), (0, _SC_LANES - _DIM)))
    q = _sc_gather(cbp, idx.reshape(n_rows // _SC_LANES, _SC_LANES), n_rows)

    loss = (1.0 + _BETA) * jnp.sum(dsum[:, 0, 0]) / (n_rows * _DIM)

    qt = _tc_transpose(q.reshape(b, h * w, _SC_LANES))
    return loss, qt.reshape(b, c, h, w)
